# baseline (speedup 1.0000x reference)
import functools

import jax
import jax.numpy as jnp
from jax import lax
from jax.experimental import pallas as pl
from jax.experimental.pallas import tpu as pltpu
from jax.experimental.pallas import tpu_sc as plsc

NC = 2
NS = 16
LANES = 16
NW = NC * NS
CH = 64
NB = 2
LOOKAHEAD = 1


def _make_embed(batch, seq_len, d_model, seq_per_w, n_sub):
    n_chunks = batch * n_sub
    mesh = plsc.VectorSubcoreMesh(
        core_axis_name="c", subcore_axis_name="s", num_cores=NC, num_subcores=NS
    )
    scratch = [
        pltpu.VMEM((batch, seq_per_w), jnp.int32),
        pltpu.VMEM((seq_per_w, d_model), jnp.float32),
        pltpu.VMEM((NB * CH, d_model), jnp.float32),
        pltpu.SemaphoreType.DMA((NB,)),
        pltpu.SemaphoreType.DMA((NB,)),
        pltpu.SemaphoreType.DMA,
        pltpu.SemaphoreType.DMA,
    ]

    @functools.partial(
        pl.kernel,
        out_type=jax.ShapeDtypeStruct((batch * seq_len, d_model), jnp.float32),
        mesh=mesh,
        scratch_types=scratch,
    )
    def embed(idx_hbm, table_hbm, pos_hbm, out_hbm,
              idx_v, pos_v, rows_v, gsems, osems, psem, isem):
        wid = lax.axis_index("s") * NC + lax.axis_index("c")
        seq0 = wid * seq_per_w

        pos_cp = pltpu.async_copy(
            pos_hbm.at[pl.ds(seq0, seq_per_w)], pos_v, psem)
        idx_cps = [
            pltpu.async_copy(
                idx_hbm.at[b, pl.ds(seq0, seq_per_w)], idx_v.at[b], isem)
            for b in range(batch)
        ]
        for cp in idx_cps:
            cp.wait()

        def chunk_pos(c):
            bi = c // n_sub
            sub = c - bi * n_sub
            return bi, sub

        def slot(c):
            return lax.rem(c, NB) * CH

        def gather(c):
            bi, sub = chunk_pos(c)
            pltpu.async_copy(
                table_hbm.at[idx_v.at[bi, pl.ds(sub * CH, CH)]],
                rows_v.at[pl.ds(slot(c), CH)], gsems.at[lax.rem(c, NB)])

        def put(c):
            bi, sub = chunk_pos(c)
            pltpu.async_copy(
                rows_v.at[pl.ds(slot(c), CH)],
                out_hbm.at[pl.ds(bi * seq_len + seq0 + sub * CH, CH)],
                osems.at[lax.rem(c, NB)])

        def drain(sem):
            pltpu.make_async_copy(
                out_hbm.at[pl.ds(0, CH)], rows_v.at[pl.ds(0, CH)], sem).wait()

        @pl.loop(0, min(LOOKAHEAD, n_chunks))
        def _prime(c):
            gather(c)
        pos_cp.wait()

        @pl.loop(0, n_chunks)
        def _chunk(c):
            pf = c + LOOKAHEAD

            @pl.when(pf < n_chunks)
            def _():
                @pl.when(pf >= NB)
                def _():
                    drain(osems.at[lax.rem(pf, NB)])
                gather(pf)

            drain(gsems.at[lax.rem(c, NB)])
            base = slot(c)
            _, sub = chunk_pos(c)

            @plsc.parallel_loop(0, CH, unroll=4)
            def _row(r):
                for j in range(d_model // LANES):
                    sl = pl.ds(j * LANES, LANES)
                    plsc.addupdate(rows_v.at[base + r, sl],
                                   pos_v[sub * CH + r, sl])

            put(c)

        @pl.loop(0, min(NB, n_chunks))
        def _drain_tail(b):
            drain(osems.at[b])

    return embed


def kernel(token_ids, table, pos_encoding):
    if token_ids.ndim == 1:
        token_ids = token_ids[None, :]
    batch, seq_len = token_ids.shape
    d_model = table.shape[1]
    assert seq_len % NW == 0
    seq_per_w = seq_len // NW
    assert seq_per_w % CH == 0
    n_sub = seq_per_w // CH
    out = _make_embed(batch, seq_len, d_model, seq_per_w, n_sub)(
        token_ids.astype(jnp.int32), table, pos_encoding)
    return out.reshape(batch, seq_len, d_model)


# speedup vs baseline: 1.0498x; 1.0498x over previous
import functools

import jax
import jax.numpy as jnp
from jax import lax
from jax.experimental import pallas as pl
from jax.experimental.pallas import tpu as pltpu
from jax.experimental.pallas import tpu_sc as plsc

NC = 2
NS = 16
LANES = 16
NW = NC * NS
CH = 32
NB = 5
LOOKAHEAD = 2


def _make_embed(batch, seq_len, d_model, seq_per_w, n_sub):
    n_chunks = batch * n_sub
    mesh = plsc.VectorSubcoreMesh(
        core_axis_name="c", subcore_axis_name="s", num_cores=NC, num_subcores=NS
    )
    scratch = [
        pltpu.VMEM((batch, seq_per_w), jnp.int32),
        pltpu.VMEM((seq_per_w, d_model), jnp.float32),
        pltpu.VMEM((NB * CH, d_model), jnp.float32),
        pltpu.SemaphoreType.DMA((NB,)),
        pltpu.SemaphoreType.DMA((NB,)),
        pltpu.SemaphoreType.DMA,
        pltpu.SemaphoreType.DMA,
    ]

    @functools.partial(
        pl.kernel,
        out_type=jax.ShapeDtypeStruct((batch * seq_len, d_model), jnp.float32),
        mesh=mesh,
        scratch_types=scratch,
    )
    def embed(idx_hbm, table_hbm, pos_hbm, out_hbm,
              idx_v, pos_v, rows_v, gsems, osems, psem, isem):
        wid = lax.axis_index("s") * NC + lax.axis_index("c")
        seq0 = wid * seq_per_w

        pos_cp = pltpu.async_copy(
            pos_hbm.at[pl.ds(seq0, seq_per_w)], pos_v, psem)
        idx_cps = [
            pltpu.async_copy(
                idx_hbm.at[b, pl.ds(seq0, seq_per_w)], idx_v.at[b], isem)
            for b in range(batch)
        ]
        for cp in idx_cps:
            cp.wait()

        def chunk_pos(c):
            bi = c // n_sub
            sub = c - bi * n_sub
            return bi, sub

        def slot(c):
            return lax.rem(c, NB) * CH

        def gather(c):
            bi, sub = chunk_pos(c)
            pltpu.async_copy(
                table_hbm.at[idx_v.at[bi, pl.ds(sub * CH, CH)]],
                rows_v.at[pl.ds(slot(c), CH)], gsems.at[lax.rem(c, NB)])

        def put(c):
            bi, sub = chunk_pos(c)
            pltpu.async_copy(
                rows_v.at[pl.ds(slot(c), CH)],
                out_hbm.at[pl.ds(bi * seq_len + seq0 + sub * CH, CH)],
                osems.at[lax.rem(c, NB)])

        def drain(sem):
            pltpu.make_async_copy(
                out_hbm.at[pl.ds(0, CH)], rows_v.at[pl.ds(0, CH)], sem).wait()

        @pl.loop(0, min(LOOKAHEAD, n_chunks))
        def _prime(c):
            gather(c)
        pos_cp.wait()

        @pl.loop(0, n_chunks)
        def _chunk(c):
            pf = c + LOOKAHEAD

            @pl.when(pf < n_chunks)
            def _():
                @pl.when(pf >= NB)
                def _():
                    drain(osems.at[lax.rem(pf, NB)])
                gather(pf)

            drain(gsems.at[lax.rem(c, NB)])
            base = slot(c)
            _, sub = chunk_pos(c)

            @plsc.parallel_loop(0, CH, unroll=4)
            def _row(r):
                for j in range(d_model // LANES):
                    sl = pl.ds(j * LANES, LANES)
                    plsc.addupdate(rows_v.at[base + r, sl],
                                   pos_v[sub * CH + r, sl])

            put(c)

        @pl.loop(0, min(NB, n_chunks))
        def _drain_tail(b):
            drain(osems.at[b])

    return embed


def kernel(token_ids, table, pos_encoding):
    if token_ids.ndim == 1:
        token_ids = token_ids[None, :]
    batch, seq_len = token_ids.shape
    d_model = table.shape[1]
    assert seq_len % NW == 0
    seq_per_w = seq_len // NW
    assert seq_per_w % CH == 0
    n_sub = seq_per_w // CH
    out = _make_embed(batch, seq_len, d_model, seq_per_w, n_sub)(
        token_ids.astype(jnp.int32), table, pos_encoding)
    return out.reshape(batch, seq_len, d_model)


# speedup vs baseline: 1.0572x; 1.0070x over previous
import functools

import jax
import jax.numpy as jnp
from jax import lax
from jax.experimental import pallas as pl
from jax.experimental.pallas import tpu as pltpu
from jax.experimental.pallas import tpu_sc as plsc

NC = 2
NS = 16
LANES = 16
NW = NC * NS
CH = 32
NB = 5
LOOKAHEAD = 2


def _make_embed(batch, seq_len, d_model, seq_per_w, n_sub):
    n_chunks = batch * n_sub
    mesh = plsc.VectorSubcoreMesh(
        core_axis_name="c", subcore_axis_name="s", num_cores=NC, num_subcores=NS
    )
    scratch = [
        pltpu.VMEM((batch, seq_per_w), jnp.int32),
        pltpu.VMEM((seq_per_w, d_model), jnp.float32),
        pltpu.VMEM((NB * CH, d_model), jnp.float32),
        pltpu.SemaphoreType.DMA((NB,)),
        pltpu.SemaphoreType.DMA((NB,)),
        pltpu.SemaphoreType.DMA,
        pltpu.SemaphoreType.DMA,
    ]

    @functools.partial(
        pl.kernel,
        out_type=jax.ShapeDtypeStruct((batch * seq_len, d_model), jnp.float32),
        mesh=mesh,
        scratch_types=scratch,
    )
    def embed(idx_hbm, table_hbm, pos_hbm, out_hbm,
              idx_v, pos_v, rows_v, gsems, osems, psem, isem):
        wid = lax.axis_index("s") * NC + lax.axis_index("c")
        seq0 = wid * seq_per_w

        pos_cp = pltpu.async_copy(
            pos_hbm.at[pl.ds(seq0, seq_per_w)], pos_v, psem)
        idx_cps = [
            pltpu.async_copy(
                idx_hbm.at[b, pl.ds(seq0, seq_per_w)], idx_v.at[b], isem)
            for b in range(batch)
        ]
        for cp in idx_cps:
            cp.wait()

        def chunk_pos(c):
            bi = c // n_sub
            sub = c - bi * n_sub
            return bi, sub

        def slot(c):
            return lax.rem(c, NB) * CH

        def gather(c):
            bi, sub = chunk_pos(c)
            pltpu.async_copy(
                table_hbm.at[idx_v.at[bi, pl.ds(sub * CH, CH)]],
                rows_v.at[pl.ds(slot(c), CH)], gsems.at[lax.rem(c, NB)])

        def put(c):
            bi, sub = chunk_pos(c)
            pltpu.async_copy(
                rows_v.at[pl.ds(slot(c), CH)],
                out_hbm.at[pl.ds(bi * seq_len + seq0 + sub * CH, CH)],
                osems.at[lax.rem(c, NB)])

        def drain(sem):
            pltpu.make_async_copy(
                out_hbm.at[pl.ds(0, CH)], rows_v.at[pl.ds(0, CH)], sem).wait()

        @pl.loop(0, min(LOOKAHEAD, n_chunks))
        def _prime(c):
            gather(c)
        pos_cp.wait()

        @pl.loop(0, n_chunks)
        def _chunk(c):
            pf = c + LOOKAHEAD

            @pl.when(pf < n_chunks)
            def _():
                @pl.when(pf >= NB)
                def _():
                    drain(osems.at[lax.rem(pf, NB)])
                gather(pf)

            drain(gsems.at[lax.rem(c, NB)])
            base = slot(c)
            _, sub = chunk_pos(c)

            @plsc.parallel_loop(0, CH, unroll=8)
            def _row(r):
                for j in range(d_model // LANES):
                    sl = pl.ds(j * LANES, LANES)
                    plsc.addupdate(rows_v.at[base + r, sl],
                                   pos_v[sub * CH + r, sl])

            put(c)

        @pl.loop(0, min(NB, n_chunks))
        def _drain_tail(b):
            drain(osems.at[b])

    return embed


def kernel(token_ids, table, pos_encoding):
    if token_ids.ndim == 1:
        token_ids = token_ids[None, :]
    batch, seq_len = token_ids.shape
    d_model = table.shape[1]
    assert seq_len % NW == 0
    seq_per_w = seq_len // NW
    assert seq_per_w % CH == 0
    n_sub = seq_per_w // CH
    out = _make_embed(batch, seq_len, d_model, seq_per_w, n_sub)(
        token_ids.astype(jnp.int32), table, pos_encoding)
    return out.reshape(batch, seq_len, d_model)
